# Initial kernel scaffold; baseline (speedup 1.0000x reference)
import jax
import jax.numpy as jnp
from jax.experimental import pallas as pl


def kernel(dec1_probs, dec2_probs, dec1_tgt, dec2_tgt, constrainer):
    raise NotImplementedError("write your pallas kernel here")


# speedup vs baseline: 1.7444x; 1.7444x over previous
import functools

import jax
import jax.numpy as jnp
from jax import lax
from jax.experimental import pallas as pl
from jax.experimental.pallas import tpu as pltpu
from jax.experimental.pallas import tpu_sc as plsc

V1 = 4096
V2 = 4096
B = 32
L = 128
N = B * L
NC = 2
NS = 16
NW = NC * NS
C = N // NW
LANES = 16


def _sc_gather_body(d1_hbm, d2_hbm, cons_hbm, t1_hbm, t2_hbm, out_hbm,
                    t1_v, t2_v, idx1_v, idx2_v, idxc_v, g1_v, g2_v, gc_v, sem):
    wid = lax.axis_index("s") * NC + lax.axis_index("c")
    base = wid * C

    pltpu.sync_copy(t1_hbm.at[pl.ds(base, C)], t1_v)
    pltpu.sync_copy(t2_hbm.at[pl.ds(base, C)], t2_v)

    for j in range(C // LANES):
        sl = pl.ds(j * LANES, LANES)
        pos = lax.iota(jnp.int32, LANES) + (base + j * LANES)
        a = t1_v[sl]
        b = t2_v[sl]
        idx1_v[sl] = pos * V1 + a
        idx2_v[sl] = pos * V2 + b
        idxc_v[sl] = a * V2 + b

    cp1 = pltpu.async_copy(d1_hbm.at[idx1_v], g1_v, sem)
    cp2 = pltpu.async_copy(d2_hbm.at[idx2_v], g2_v, sem)
    cp3 = pltpu.async_copy(cons_hbm.at[idxc_v], gc_v, sem)
    cp1.wait()
    cp2.wait()
    cp3.wait()

    pltpu.sync_copy(g1_v, out_hbm.at[pl.ds(base, C)])
    pltpu.sync_copy(g2_v, out_hbm.at[pl.ds(N + base, C)])
    pltpu.sync_copy(gc_v, out_hbm.at[pl.ds(2 * N + base, C)])


_sc_gather = functools.partial(
    pl.kernel,
    out_type=jax.ShapeDtypeStruct((3 * N,), jnp.float32),
    mesh=plsc.VectorSubcoreMesh(
        core_axis_name="c", subcore_axis_name="s",
        num_cores=NC, num_subcores=NS),
    scratch_types=[
        pltpu.VMEM((C,), jnp.int32),
        pltpu.VMEM((C,), jnp.int32),
        pltpu.VMEM((C,), jnp.int32),
        pltpu.VMEM((C,), jnp.int32),
        pltpu.VMEM((C,), jnp.int32),
        pltpu.VMEM((C,), jnp.float32),
        pltpu.VMEM((C,), jnp.float32),
        pltpu.VMEM((C,), jnp.float32),
        pltpu.SemaphoreType.DMA,
    ],
)(_sc_gather_body)


def _tc_reduce_body(g_ref, out_ref):
    g1 = g_ref[pl.ds(0, N)]
    g2 = g_ref[pl.ds(N, N)]
    gc = jnp.clip(g_ref[pl.ds(2 * N, N)], 0.0, 1.0)
    s = jnp.sum(jnp.log(g1 * gc)) + jnp.sum(jnp.log(g2 * gc))
    out_ref[0, 0] = -s / jnp.float32(N)


_tc_reduce = pl.pallas_call(
    _tc_reduce_body,
    out_shape=jax.ShapeDtypeStruct((1, 1), jnp.float32),
    out_specs=pl.BlockSpec(memory_space=pltpu.SMEM),
)


def kernel(dec1_probs, dec2_probs, dec1_tgt, dec2_tgt, constrainer):
    d1 = dec1_probs.reshape(N * V1)
    d2 = dec2_probs.reshape(N * V2)
    cons = constrainer.reshape(V1 * V2)
    t1 = dec1_tgt.reshape(N)
    t2 = dec2_tgt.reshape(N)
    gathered = _sc_gather(d1, d2, cons, t1, t2)
    return _tc_reduce(gathered)[0, 0]
